# Initial kernel scaffold; baseline (speedup 1.0000x reference)
import jax
import jax.numpy as jnp
from jax.experimental import pallas as pl


def kernel(x, edge_index, W_self1, W_neigh1, b1, W_self2, W_neigh2, b2):
    raise NotImplementedError("write your pallas kernel here")


# speedup vs baseline: 13.8164x; 13.8164x over previous
import functools

import jax
import jax.numpy as jnp
from jax import lax
from jax.experimental import pallas as pl
from jax.experimental.pallas import tpu as pltpu
from jax.experimental.pallas import tpu_sc as plsc

NC = 2
NS = 16
NW = NC * NS
CH = 128
BM = 512


def _sc_aggregate(with_deg: bool, n_acc: int, k_chunks: int):
    n_out = 2 + (2 if with_deg else 0)
    out_type = tuple(
        jax.ShapeDtypeStruct((n_acc, 16), jnp.float32) for _ in range(n_out))
    scratch = [
        pltpu.VMEM((k_chunks, CH), jnp.int32),
        pltpu.VMEM((k_chunks, CH), jnp.int32),
        pltpu.VMEM((CH, 16), jnp.float32),
        pltpu.VMEM((CH, 16), jnp.float32),
        pltpu.VMEM((CH, 16), jnp.float32),
        pltpu.VMEM_SHARED((n_acc, 16), jnp.float32),
        pltpu.SemaphoreType.DMA,
        pltpu.SemaphoreType.DMA,
    ]
    if with_deg:
        scratch.insert(6, pltpu.VMEM_SHARED((n_acc, 16), jnp.float32))

    mesh = plsc.VectorSubcoreMesh(core_axis_name="c", subcore_axis_name="s")

    def body(p_hbm, src_hbm, dst_hbm, zeros_hbm, ones_hbm, *rest):
        if with_deg:
            (agg_a, agg_b, deg_a, deg_b,
             idx_s, idx_d, rows0, rows1, ones_v, agg_sh, deg_sh,
             sem0, sem1) = rest
        else:
            (agg_a, agg_b,
             idx_s, idx_d, rows0, rows1, ones_v, agg_sh,
             sem0, sem1) = rest
        c = lax.axis_index("c")
        s = lax.axis_index("s")
        wid = s * NC + c

        rps = n_acc // NS
        zb = s * rps
        pltpu.sync_copy(zeros_hbm.at[pl.ds(zb, rps)],
                        agg_sh.at[pl.ds(zb, rps)])
        if with_deg:
            pltpu.sync_copy(zeros_hbm.at[pl.ds(zb, rps)],
                            deg_sh.at[pl.ds(zb, rps)])
            pltpu.sync_copy(ones_hbm, ones_v)
        pltpu.sync_copy(src_hbm.at[pl.ds(wid * k_chunks, k_chunks)], idx_s)
        pltpu.sync_copy(dst_hbm.at[pl.ds(wid * k_chunks, k_chunks)], idx_d)
        plsc.subcore_barrier()

        bufs = (rows0, rows1)
        sems = (sem0, sem1)
        pltpu.async_copy(p_hbm.at[idx_s.at[0]], rows0, sem0)

        def chunk_body(j, carry):
            del carry
            buf_cur = j % 2
            for b in range(2):
                @pl.when(buf_cur == b)
                def _():
                    nxt = 1 - b
                    @pl.when(j + 1 < k_chunks)
                    def _():
                        pltpu.async_copy(p_hbm.at[idx_s.at[j + 1]],
                                         bufs[nxt], sems[nxt])
                    pltpu.make_async_copy(p_hbm.at[idx_s.at[j]],
                                          bufs[b], sems[b]).wait()
                    pltpu.sync_copy(bufs[b], agg_sh.at[idx_d.at[j]], add=True)
                    if with_deg:
                        pltpu.sync_copy(ones_v, deg_sh.at[idx_d.at[j]],
                                        add=True)
            return 0

        lax.fori_loop(0, k_chunks, chunk_body, 0)
        plsc.subcore_barrier()

        ob = s * rps
        sl = pl.ds(ob, rps)
        @pl.when(c == 0)
        def _():
            pltpu.sync_copy(agg_sh.at[sl], agg_a.at[sl])
            if with_deg:
                pltpu.sync_copy(deg_sh.at[sl], deg_a.at[sl])
        @pl.when(c == 1)
        def _():
            pltpu.sync_copy(agg_sh.at[sl], agg_b.at[sl])
            if with_deg:
                pltpu.sync_copy(deg_sh.at[sl], deg_b.at[sl])

    return pl.kernel(body, out_type=out_type, mesh=mesh,
                     scratch_types=scratch,
                     compiler_params=pltpu.CompilerParams(
                         use_tc_tiling_on_sc=False))


def _tc1(x, w_self, w_neigh, b, n_pad):
    grid = (n_pad // BM,)
    d = x.shape[1]

    def body(x_ref, ws_ref, wn_ref, b_ref, s_ref, p_ref):
        xb = x_ref[...]
        s_ref[...] = jnp.dot(xb, ws_ref[...],
                             preferred_element_type=jnp.float32) + b_ref[...]
        p_ref[...] = jnp.dot(xb, wn_ref[...],
                             preferred_element_type=jnp.float32)

    return pl.pallas_call(
        body,
        grid=grid,
        in_specs=[
            pl.BlockSpec((BM, d), lambda i: (i, 0)),
            pl.BlockSpec((d, 16), lambda i: (0, 0)),
            pl.BlockSpec((d, 16), lambda i: (0, 0)),
            pl.BlockSpec((1, 16), lambda i: (0, 0)),
        ],
        out_specs=[
            pl.BlockSpec((BM, 16), lambda i: (i, 0)),
            pl.BlockSpec((BM, 16), lambda i: (i, 0)),
        ],
        out_shape=[
            jax.ShapeDtypeStruct((n_pad, 16), jnp.float32),
            jax.ShapeDtypeStruct((n_pad, 16), jnp.float32),
        ],
    )(x, w_self, w_neigh, b)


def _tc2(s1, agg_a, agg_b, deg_a, deg_b, w_self2, w_neigh2, b2, n_pad):
    grid = (n_pad // BM,)
    blk = pl.BlockSpec((BM, 16), lambda i: (i, 0))
    full = pl.BlockSpec((16, 16), lambda i: (0, 0))

    def body(s1_ref, aa_ref, ab_ref, da_ref, db_ref, ws_ref, wn_ref, b_ref,
             s2_ref, p2_ref, rd_ref):
        deg = da_ref[...] + db_ref[...]
        rdeg = 1.0 / jnp.maximum(deg, 1.0)
        h = jnp.maximum(s1_ref[...] + (aa_ref[...] + ab_ref[...]) * rdeg, 0.0)
        s2_ref[...] = jnp.dot(h, ws_ref[...],
                              preferred_element_type=jnp.float32) + b_ref[...]
        p2_ref[...] = jnp.dot(h, wn_ref[...],
                              preferred_element_type=jnp.float32)
        rd_ref[...] = rdeg

    return pl.pallas_call(
        body,
        grid=grid,
        in_specs=[blk, blk, blk, blk, blk, full, full,
                  pl.BlockSpec((1, 16), lambda i: (0, 0))],
        out_specs=[blk, blk, blk],
        out_shape=[
            jax.ShapeDtypeStruct((n_pad, 16), jnp.float32),
            jax.ShapeDtypeStruct((n_pad, 16), jnp.float32),
            jax.ShapeDtypeStruct((n_pad, 16), jnp.float32),
        ],
    )(s1, agg_a, agg_b, deg_a, deg_b, w_self2, w_neigh2, b2)


def _tc3(s2, agg_a, agg_b, rdeg, n_pad):
    grid = (n_pad // BM,)
    blk = pl.BlockSpec((BM, 16), lambda i: (i, 0))

    def body(s2_ref, aa_ref, ab_ref, rd_ref, o_ref):
        o_ref[...] = s2_ref[...] + (aa_ref[...] + ab_ref[...]) * rd_ref[...]

    return pl.pallas_call(
        body,
        grid=grid,
        in_specs=[blk, blk, blk, blk],
        out_specs=blk,
        out_shape=jax.ShapeDtypeStruct((n_pad, 16), jnp.float32),
    )(s2, agg_a, agg_b, rdeg)


def kernel(x, edge_index, W_self1, W_neigh1, b1, W_self2, W_neigh2, b2):
    n, d = x.shape
    e = edge_index.shape[1]

    n_pad = ((n + 16 + BM - 1) // BM) * BM
    assert n_pad % (8 * NS) == 0

    k_chunks = -(-e // (NW * CH))
    k_chunks = ((k_chunks + 7) // 8) * 8
    ept = k_chunks * CH
    e_pad = ept * NW
    src = jnp.concatenate(
        [edge_index[0], jnp.zeros((e_pad - e,), jnp.int32)])
    dst = jnp.concatenate(
        [edge_index[1], jnp.full((e_pad - e,), n, jnp.int32)])
    src2d = src.reshape(NW * k_chunks, CH)
    dst2d = dst.reshape(NW * k_chunks, CH)

    zeros = jnp.zeros((n_pad, 16), jnp.float32)
    ones = jnp.ones((CH, 16), jnp.float32)

    x_p = jnp.pad(x, ((0, n_pad - n), (0, 0)))
    b1r = b1.reshape(1, 16)
    b2r = b2.reshape(1, 16)

    s1, p1 = _tc1(x_p, W_self1, W_neigh1, b1r, n_pad)
    sc1 = _sc_aggregate(True, n_pad, k_chunks)
    agg1a, agg1b, deg_a, deg_b = sc1(p1, src2d, dst2d, zeros, ones)
    s2, p2, rdeg = _tc2(s1, agg1a, agg1b, deg_a, deg_b,
                        W_self2, W_neigh2, b2r, n_pad)

    sc2 = _sc_aggregate(False, n_pad, k_chunks)
    agg2a, agg2b = sc2(p2, src2d, dst2d, zeros, ones)
    out = _tc3(s2, agg2a, agg2b, rdeg, n_pad)
    return out[:n]


# speedup vs baseline: 14.4482x; 1.0457x over previous
import functools

import jax
import jax.numpy as jnp
from jax import lax
from jax.experimental import pallas as pl
from jax.experimental.pallas import tpu as pltpu
from jax.experimental.pallas import tpu_sc as plsc

NC = 2
NS = 16
NW = NC * NS
CH = 128
BM = 512


def _sc_aggregate(with_deg: bool, n_acc: int, k_chunks: int):
    n_out = 2 + (2 if with_deg else 0)
    out_type = tuple(
        jax.ShapeDtypeStruct((n_acc, 16), jnp.float32) for _ in range(n_out))
    NB = 8
    LA = 4
    scratch = [
        pltpu.VMEM((k_chunks, CH), jnp.int32),
        pltpu.VMEM((k_chunks, CH), jnp.int32),
        pltpu.VMEM((NB, CH, 16), jnp.float32),
        pltpu.VMEM((CH, 16), jnp.float32),
        pltpu.VMEM_SHARED((n_acc, 16), jnp.float32),
        pltpu.SemaphoreType.DMA((NB,)),
        pltpu.SemaphoreType.DMA((NB,)),
    ]
    if with_deg:
        scratch.insert(5, pltpu.VMEM_SHARED((n_acc, 16), jnp.float32))

    mesh = plsc.VectorSubcoreMesh(core_axis_name="c", subcore_axis_name="s")

    def body(p_hbm, src_hbm, dst_hbm, zeros_hbm, ones_hbm, *rest):
        if with_deg:
            (agg_a, agg_b, deg_a, deg_b,
             idx_s, idx_d, rows, ones_v, agg_sh, deg_sh,
             gsem, ssem) = rest
        else:
            (agg_a, agg_b,
             idx_s, idx_d, rows, ones_v, agg_sh,
             gsem, ssem) = rest
        c = lax.axis_index("c")
        s = lax.axis_index("s")
        wid = s * NC + c

        rps = n_acc // NS
        zb = s * rps
        pltpu.sync_copy(zeros_hbm.at[pl.ds(zb, rps)],
                        agg_sh.at[pl.ds(zb, rps)])
        if with_deg:
            pltpu.sync_copy(zeros_hbm.at[pl.ds(zb, rps)],
                            deg_sh.at[pl.ds(zb, rps)])
            pltpu.sync_copy(ones_hbm, ones_v)
        pltpu.sync_copy(src_hbm.at[pl.ds(wid * k_chunks, k_chunks)], idx_s)
        pltpu.sync_copy(dst_hbm.at[pl.ds(wid * k_chunks, k_chunks)], idx_d)
        plsc.subcore_barrier()

        for j0 in range(LA):
            pltpu.async_copy(p_hbm.at[idx_s.at[j0]], rows.at[j0],
                             gsem.at[j0])

        def chunk_body(j, carry):
            del carry
            jb = j % NB
            for b in range(NB):
                @pl.when(jb == b)
                def _():
                    bp = (b + LA) % NB
                    @pl.when(j + LA < k_chunks)
                    def _():
                        @pl.when(j >= NB - LA)
                        def _():
                            pltpu.make_async_copy(
                                rows.at[bp], agg_sh.at[idx_d.at[0]],
                                ssem.at[bp]).wait()
                        pltpu.async_copy(p_hbm.at[idx_s.at[j + LA]],
                                         rows.at[bp], gsem.at[bp])
                    pltpu.make_async_copy(p_hbm.at[idx_s.at[j]],
                                          rows.at[b], gsem.at[b]).wait()
                    pltpu.async_copy(rows.at[b], agg_sh.at[idx_d.at[j]],
                                     ssem.at[b], add=True)
                    if with_deg:
                        pltpu.sync_copy(ones_v, deg_sh.at[idx_d.at[j]],
                                        add=True)
            return 0

        lax.fori_loop(0, k_chunks, chunk_body, 0)
        for jj in range(max(0, k_chunks - NB), k_chunks):
            b = jj % NB
            pltpu.make_async_copy(rows.at[b], agg_sh.at[idx_d.at[0]],
                                  ssem.at[b]).wait()
        plsc.subcore_barrier()

        ob = s * rps
        sl = pl.ds(ob, rps)
        @pl.when(c == 0)
        def _():
            pltpu.sync_copy(agg_sh.at[sl], agg_a.at[sl])
            if with_deg:
                pltpu.sync_copy(deg_sh.at[sl], deg_a.at[sl])
        @pl.when(c == 1)
        def _():
            pltpu.sync_copy(agg_sh.at[sl], agg_b.at[sl])
            if with_deg:
                pltpu.sync_copy(deg_sh.at[sl], deg_b.at[sl])

    return pl.kernel(body, out_type=out_type, mesh=mesh,
                     scratch_types=scratch,
                     compiler_params=pltpu.CompilerParams(
                         use_tc_tiling_on_sc=False))


def _tc1(x, w_self, w_neigh, b, n_blocks):
    grid = (n_blocks,)
    n, d = x.shape

    def body(x_ref, ws_ref, wn_ref, b_ref, s_ref, p_ref):
        xb = x_ref[...]
        s_ref[...] = jnp.dot(xb, ws_ref[...],
                             preferred_element_type=jnp.float32) + b_ref[...]
        p_ref[...] = jnp.dot(xb, wn_ref[...],
                             preferred_element_type=jnp.float32)

    return pl.pallas_call(
        body,
        grid=grid,
        in_specs=[
            pl.BlockSpec((BM, d), lambda i: (i, 0)),
            pl.BlockSpec((d, 16), lambda i: (0, 0)),
            pl.BlockSpec((d, 16), lambda i: (0, 0)),
            pl.BlockSpec((1, 16), lambda i: (0, 0)),
        ],
        out_specs=[
            pl.BlockSpec((BM, 16), lambda i: (i, 0)),
            pl.BlockSpec((BM, 16), lambda i: (i, 0)),
        ],
        out_shape=[
            jax.ShapeDtypeStruct((n, 16), jnp.float32),
            jax.ShapeDtypeStruct((n, 16), jnp.float32),
        ],
    )(x, w_self, w_neigh, b)


def _tc2(s1, agg_a, agg_b, deg_a, deg_b, w_self2, w_neigh2, b2, n_blocks):
    grid = (n_blocks,)
    n = s1.shape[0]
    blk = pl.BlockSpec((BM, 16), lambda i: (i, 0))
    full = pl.BlockSpec((16, 16), lambda i: (0, 0))

    def body(s1_ref, aa_ref, ab_ref, da_ref, db_ref, ws_ref, wn_ref, b_ref,
             s2_ref, p2_ref, rd_ref):
        deg = da_ref[...] + db_ref[...]
        rdeg = 1.0 / jnp.maximum(deg, 1.0)
        h = jnp.maximum(s1_ref[...] + (aa_ref[...] + ab_ref[...]) * rdeg, 0.0)
        s2_ref[...] = jnp.dot(h, ws_ref[...],
                              preferred_element_type=jnp.float32) + b_ref[...]
        p2_ref[...] = jnp.dot(h, wn_ref[...],
                              preferred_element_type=jnp.float32)
        rd_ref[...] = rdeg

    return pl.pallas_call(
        body,
        grid=grid,
        in_specs=[blk, blk, blk, blk, blk, full, full,
                  pl.BlockSpec((1, 16), lambda i: (0, 0))],
        out_specs=[blk, blk, blk],
        out_shape=[
            jax.ShapeDtypeStruct((n, 16), jnp.float32),
            jax.ShapeDtypeStruct((n, 16), jnp.float32),
            jax.ShapeDtypeStruct((n, 16), jnp.float32),
        ],
    )(s1, agg_a, agg_b, deg_a, deg_b, w_self2, w_neigh2, b2)


def _tc3(s2, agg_a, agg_b, rdeg, n_blocks):
    grid = (n_blocks,)
    n = s2.shape[0]
    blk = pl.BlockSpec((BM, 16), lambda i: (i, 0))

    def body(s2_ref, aa_ref, ab_ref, rd_ref, o_ref):
        o_ref[...] = s2_ref[...] + (aa_ref[...] + ab_ref[...]) * rd_ref[...]

    return pl.pallas_call(
        body,
        grid=grid,
        in_specs=[blk, blk, blk, blk],
        out_specs=blk,
        out_shape=jax.ShapeDtypeStruct((n, 16), jnp.float32),
    )(s2, agg_a, agg_b, rdeg)


def kernel(x, edge_index, W_self1, W_neigh1, b1, W_self2, W_neigh2, b2):
    n, d = x.shape
    e = edge_index.shape[1]

    n_acc = ((n + 16 + BM - 1) // BM) * BM
    assert n_acc % (8 * NS) == 0
    n_blocks = -(-n // BM)

    k_chunks = -(-e // (NW * CH))
    k_chunks = ((k_chunks + 7) // 8) * 8
    ept = k_chunks * CH
    e_pad = ept * NW
    src = jnp.concatenate(
        [edge_index[0], jnp.zeros((e_pad - e,), jnp.int32)])
    dst = jnp.concatenate(
        [edge_index[1], jnp.full((e_pad - e,), n, jnp.int32)])
    src2d = src.reshape(NW * k_chunks, CH)
    dst2d = dst.reshape(NW * k_chunks, CH)

    zeros = jnp.zeros((n_acc, 16), jnp.float32)
    ones = jnp.ones((CH, 16), jnp.float32)

    b1r = b1.reshape(1, 16)
    b2r = b2.reshape(1, 16)

    s1, p1 = _tc1(x, W_self1, W_neigh1, b1r, n_blocks)
    sc1 = _sc_aggregate(True, n_acc, k_chunks)
    agg1a, agg1b, deg_a, deg_b = sc1(p1, src2d, dst2d, zeros, ones)
    s2, p2, rdeg = _tc2(s1, agg1a, agg1b, deg_a, deg_b,
                        W_self2, W_neigh2, b2r, n_blocks)

    sc2 = _sc_aggregate(False, n_acc, k_chunks)
    agg2a, agg2b = sc2(p2, src2d, dst2d, zeros, ones)
    return _tc3(s2, agg2a, agg2b, rdeg, n_blocks)


# speedup vs baseline: 17.4863x; 1.2103x over previous
import functools

import jax
import jax.numpy as jnp
from jax import lax
from jax.experimental import pallas as pl
from jax.experimental.pallas import tpu as pltpu
from jax.experimental.pallas import tpu_sc as plsc

NC = 2
NS = 16
NW = NC * NS
CH = 128
BM = 2048


def _sc_aggregate(with_deg: bool, n_acc: int, k_chunks: int):
    n_out = 2 + (2 if with_deg else 0)
    out_type = tuple(
        jax.ShapeDtypeStruct((n_acc, 16), jnp.float32) for _ in range(n_out))
    NB = 8
    LA = 4
    scratch = [
        pltpu.VMEM((k_chunks, CH), jnp.int32),
        pltpu.VMEM((k_chunks, CH), jnp.int32),
        pltpu.VMEM((NB, CH, 16), jnp.float32),
        pltpu.VMEM((CH, 16), jnp.float32),
        pltpu.VMEM_SHARED((n_acc, 16), jnp.float32),
        pltpu.SemaphoreType.DMA((NB,)),
        pltpu.SemaphoreType.DMA((NB,)),
    ]
    if with_deg:
        scratch.insert(5, pltpu.VMEM_SHARED((n_acc, 16), jnp.float32))

    mesh = plsc.VectorSubcoreMesh(core_axis_name="c", subcore_axis_name="s")

    def body(p_hbm, src_hbm, dst_hbm, zeros_hbm, ones_hbm, *rest):
        if with_deg:
            (agg_a, agg_b, deg_a, deg_b,
             idx_s, idx_d, rows, ones_v, agg_sh, deg_sh,
             gsem, ssem) = rest
        else:
            (agg_a, agg_b,
             idx_s, idx_d, rows, ones_v, agg_sh,
             gsem, ssem) = rest
        c = lax.axis_index("c")
        s = lax.axis_index("s")
        wid = s * NC + c

        rps = n_acc // NS
        zb = s * rps
        pltpu.sync_copy(zeros_hbm.at[pl.ds(zb, rps)],
                        agg_sh.at[pl.ds(zb, rps)])
        if with_deg:
            pltpu.sync_copy(zeros_hbm.at[pl.ds(zb, rps)],
                            deg_sh.at[pl.ds(zb, rps)])
            pltpu.sync_copy(ones_hbm, ones_v)
        pltpu.sync_copy(src_hbm.at[pl.ds(wid * k_chunks, k_chunks)], idx_s)
        pltpu.sync_copy(dst_hbm.at[pl.ds(wid * k_chunks, k_chunks)], idx_d)
        plsc.subcore_barrier()

        for j0 in range(LA):
            pltpu.async_copy(p_hbm.at[idx_s.at[j0]], rows.at[j0],
                             gsem.at[j0])

        def chunk_body(j, carry):
            del carry
            jb = j % NB
            for b in range(NB):
                @pl.when(jb == b)
                def _():
                    bp = (b + LA) % NB
                    @pl.when(j + LA < k_chunks)
                    def _():
                        @pl.when(j >= NB - LA)
                        def _():
                            pltpu.make_async_copy(
                                rows.at[bp], agg_sh.at[idx_d.at[0]],
                                ssem.at[bp]).wait()
                        pltpu.async_copy(p_hbm.at[idx_s.at[j + LA]],
                                         rows.at[bp], gsem.at[bp])
                    pltpu.make_async_copy(p_hbm.at[idx_s.at[j]],
                                          rows.at[b], gsem.at[b]).wait()
                    pltpu.async_copy(rows.at[b], agg_sh.at[idx_d.at[j]],
                                     ssem.at[b], add=True)
                    if with_deg:
                        pltpu.sync_copy(ones_v, deg_sh.at[idx_d.at[j]],
                                        add=True)
            return 0

        lax.fori_loop(0, k_chunks, chunk_body, 0)
        for jj in range(max(0, k_chunks - NB), k_chunks):
            b = jj % NB
            pltpu.make_async_copy(rows.at[b], agg_sh.at[idx_d.at[0]],
                                  ssem.at[b]).wait()
        plsc.subcore_barrier()

        ob = s * rps
        sl = pl.ds(ob, rps)
        @pl.when(c == 0)
        def _():
            pltpu.sync_copy(agg_sh.at[sl], agg_a.at[sl])
            if with_deg:
                pltpu.sync_copy(deg_sh.at[sl], deg_a.at[sl])
        @pl.when(c == 1)
        def _():
            pltpu.sync_copy(agg_sh.at[sl], agg_b.at[sl])
            if with_deg:
                pltpu.sync_copy(deg_sh.at[sl], deg_b.at[sl])

    return pl.kernel(body, out_type=out_type, mesh=mesh,
                     scratch_types=scratch,
                     compiler_params=pltpu.CompilerParams(
                         use_tc_tiling_on_sc=False))


def _tc1(x, w_self, w_neigh, b, n_blocks):
    grid = (n_blocks,)
    n, d = x.shape

    def body(x_ref, ws_ref, wn_ref, b_ref, s_ref, p_ref):
        xb = x_ref[...]
        s_ref[...] = jnp.dot(xb, ws_ref[...],
                             preferred_element_type=jnp.float32) + b_ref[...]
        p_ref[...] = jnp.dot(xb, wn_ref[...],
                             preferred_element_type=jnp.float32)

    return pl.pallas_call(
        body,
        grid=grid,
        in_specs=[
            pl.BlockSpec((BM, d), lambda i: (i, 0)),
            pl.BlockSpec((d, 16), lambda i: (0, 0)),
            pl.BlockSpec((d, 16), lambda i: (0, 0)),
            pl.BlockSpec((1, 16), lambda i: (0, 0)),
        ],
        out_specs=[
            pl.BlockSpec((BM, 16), lambda i: (i, 0)),
            pl.BlockSpec((BM, 16), lambda i: (i, 0)),
        ],
        out_shape=[
            jax.ShapeDtypeStruct((n, 16), jnp.float32),
            jax.ShapeDtypeStruct((n, 16), jnp.float32),
        ],
    )(x, w_self, w_neigh, b)


def _tc2(s1, agg_a, agg_b, deg_a, deg_b, w_self2, w_neigh2, b2, n_blocks):
    grid = (n_blocks,)
    n = s1.shape[0]
    blk = pl.BlockSpec((BM, 16), lambda i: (i, 0))
    full = pl.BlockSpec((16, 16), lambda i: (0, 0))

    def body(s1_ref, aa_ref, ab_ref, da_ref, db_ref, ws_ref, wn_ref, b_ref,
             s2_ref, p2_ref, rd_ref):
        deg = da_ref[...] + db_ref[...]
        rdeg = 1.0 / jnp.maximum(deg, 1.0)
        h = jnp.maximum(s1_ref[...] + (aa_ref[...] + ab_ref[...]) * rdeg, 0.0)
        s2_ref[...] = jnp.dot(h, ws_ref[...],
                              preferred_element_type=jnp.float32) + b_ref[...]
        p2_ref[...] = jnp.dot(h, wn_ref[...],
                              preferred_element_type=jnp.float32)
        rd_ref[...] = rdeg

    return pl.pallas_call(
        body,
        grid=grid,
        in_specs=[blk, blk, blk, blk, blk, full, full,
                  pl.BlockSpec((1, 16), lambda i: (0, 0))],
        out_specs=[blk, blk, blk],
        out_shape=[
            jax.ShapeDtypeStruct((n, 16), jnp.float32),
            jax.ShapeDtypeStruct((n, 16), jnp.float32),
            jax.ShapeDtypeStruct((n, 16), jnp.float32),
        ],
    )(s1, agg_a, agg_b, deg_a, deg_b, w_self2, w_neigh2, b2)


def _tc3(s2, agg_a, agg_b, rdeg, n_blocks):
    grid = (n_blocks,)
    n = s2.shape[0]
    blk = pl.BlockSpec((BM, 16), lambda i: (i, 0))

    def body(s2_ref, aa_ref, ab_ref, rd_ref, o_ref):
        o_ref[...] = s2_ref[...] + (aa_ref[...] + ab_ref[...]) * rd_ref[...]

    return pl.pallas_call(
        body,
        grid=grid,
        in_specs=[blk, blk, blk, blk],
        out_specs=blk,
        out_shape=jax.ShapeDtypeStruct((n, 16), jnp.float32),
    )(s2, agg_a, agg_b, rdeg)


def kernel(x, edge_index, W_self1, W_neigh1, b1, W_self2, W_neigh2, b2):
    n, d = x.shape
    e = edge_index.shape[1]

    n_acc = ((n + 16 + 127) // 128) * 128
    assert n_acc % (8 * NS) == 0
    n_blocks = -(-n // BM)

    k_chunks = -(-e // (NW * CH))
    k_chunks = ((k_chunks + 7) // 8) * 8
    ept = k_chunks * CH
    e_pad = ept * NW
    pad_dst = n + jnp.arange(e_pad - e, dtype=jnp.int32) % (n_acc - n)
    src = jnp.concatenate(
        [edge_index[0], jnp.zeros((e_pad - e,), jnp.int32)])
    dst = jnp.concatenate([edge_index[1], pad_dst])
    src2d = src.reshape(NW * k_chunks, CH)
    dst2d = dst.reshape(NW * k_chunks, CH)

    zeros = jnp.zeros((n_acc, 16), jnp.float32)
    ones = jnp.ones((CH, 16), jnp.float32)

    b1r = b1.reshape(1, 16)
    b2r = b2.reshape(1, 16)

    s1, p1 = _tc1(x, W_self1, W_neigh1, b1r, n_blocks)
    sc1 = _sc_aggregate(True, n_acc, k_chunks)
    agg1a, agg1b, deg_a, deg_b = sc1(p1, src2d, dst2d, zeros, ones)
    s2, p2, rdeg = _tc2(s1, agg1a, agg1b, deg_a, deg_b,
                        W_self2, W_neigh2, b2r, n_blocks)

    sc2 = _sc_aggregate(False, n_acc, k_chunks)
    agg2a, agg2b = sc2(p2, src2d, dst2d, zeros, ones)
    return _tc3(s2, agg2a, agg2b, rdeg, n_blocks)
